# baseline (speedup 1.0000x reference)
import functools

import jax
import jax.numpy as jnp
import numpy as np
from jax import lax
from jax.experimental import pallas as pl
from jax.experimental.pallas import tpu as pltpu
from jax.experimental.pallas import tpu_sc as plsc

_N = 10000
_H = 128
_OUT = 16
_R = 16
_E = 320000

_NC = 2
_NS = 16
_NW = _NC * _NS
_EP = 10240
_E_PAD = _NW * _EP
_C = 128
_NCHUNK = _EP // _C
_NPAD = 10240
_RPT = _NPAD // _NS
_ZR = 128


def _sc_pass_body(mode, bf16_tab, d, table, metam, normm, out,
                  slab_v, rows_v, scat_v, norm_v, acc_ref, sem):
    cid = lax.axis_index("c")
    sid = lax.axis_index("s")
    wid = sid * _NC + cid
    ng = d // 16

    pltpu.sync_copy(normm.at[wid], norm_v)

    def _zero_row(r, _):
        for g in range(ng):
            scat_v[r, pl.ds(g * 16, 16)] = jnp.zeros((16,), jnp.float32)
        return 0
    lax.fori_loop(0, _ZR, _zero_row, 0)

    for j in range(_RPT // _ZR):
        pltpu.sync_copy(scat_v.at[pl.ds(0, _ZR)],
                        acc_ref.at[pl.ds(sid * _RPT + j * _ZR, _ZR)])
    plsc.subcore_barrier()

    def _chunk(i, _):
        pltpu.sync_copy(metam.at[wid, i], slab_v)

        for g in range(8):
            sl = pl.ds(g * 16, 16)
            s = slab_v[0, sl]
            rl = slab_v[1, sl]
            if mode == 0:
                slab_v[0, sl] = rl * _N + s
            else:
                slab_v[0, sl] = s * _R + rl

        pltpu.async_copy(table.at[slab_v.at[0]], rows_v, sem).wait()

        def _scale(q, _):
            nv16 = norm_v[pl.ds(i * _C + q * 16, 16)]
            for l in range(16):
                nv = jnp.full((16,), nv16[l], jnp.float32)
                e = q * 16 + l
                if bf16_tab:
                    for g in range(d // 32):
                        vi = plsc.bitcast(rows_v[e, pl.ds(g * 32, 32)],
                                          jnp.int32)
                        lo = plsc.bitcast(vi << 16, jnp.float32)
                        hi = plsc.bitcast(
                            vi & jnp.int32(-65536), jnp.float32)
                        scat_v[e, pl.ds(g * 32, 16)] = lo * nv
                        scat_v[e, pl.ds(g * 32 + 16, 16)] = hi * nv
                else:
                    for g in range(ng):
                        sl = pl.ds(g * 16, 16)
                        scat_v[e, sl] = rows_v[e, sl] * nv
            return 0
        lax.fori_loop(0, _C // 16, _scale, 0)

        pltpu.sync_copy(scat_v, acc_ref.at[slab_v.at[2]], add=True)
        return 0
    lax.fori_loop(0, _NCHUNK, _chunk, 0)

    plsc.subcore_barrier()
    pltpu.sync_copy(acc_ref.at[pl.ds(sid * _RPT, _RPT)],
                    out.at[cid, pl.ds(sid * _RPT, _RPT)])


def _sc_pass(table, metam, normm, mode):
    d = table.shape[1]
    bf16_tab = table.dtype == jnp.bfloat16
    body = functools.partial(_sc_pass_body, mode, bf16_tab, d)
    return pl.kernel(
        body,
        out_type=jax.ShapeDtypeStruct((_NC, _NPAD, d), jnp.float32),
        mesh=plsc.VectorSubcoreMesh(core_axis_name="c", subcore_axis_name="s"),
        scratch_types=[
            pltpu.VMEM((3, _C), jnp.int32),
            pltpu.VMEM((_C, d), table.dtype),
            pltpu.VMEM((_C, d), jnp.float32),
            pltpu.VMEM((_EP,), jnp.float32),
            pltpu.VMEM_SHARED((_NPAD, d), jnp.float32),
            pltpu.SemaphoreType.DMA,
        ],
        compiler_params=pltpu.CompilerParams(use_tc_tiling_on_sc=False, needs_layout_passes=False),
    )(table, metam, normm)


_PERM = tuple(
    (np.arange(0, 128, 32)[:, None]
     + np.arange(32).reshape(2, 16).T.reshape(-1)[None, :]).reshape(-1))


def _mm_body(out_dtype, p_ref, w_ref, o_ref):
    h = jnp.maximum(p_ref[0] + p_ref[1], 0.0)
    o_ref[...] = jnp.dot(
        h, w_ref[...], preferred_element_type=jnp.float32).astype(out_dtype)


def _tc_relu_matmul(p, w, out_dtype=jnp.float32):
    n = p.shape[1]
    f = w.shape[1]
    rb = 400
    fb = min(f, 512)
    grid = (n // rb, f // fb)
    return pl.pallas_call(
        functools.partial(_mm_body, out_dtype),
        grid=grid,
        in_specs=[
            pl.BlockSpec((2, rb, _H), lambda i, j: (0, i, 0)),
            pl.BlockSpec((_H, fb), lambda i, j: (0, j)),
        ],
        out_specs=pl.BlockSpec((rb, fb), lambda i, j: (i, j)),
        out_shape=jax.ShapeDtypeStruct((n, f), out_dtype),
    )(p, w)


def _softmax_body(p_ref, o_ref):
    x = p_ref[0] + p_ref[1]
    m = jnp.max(x, axis=1, keepdims=True)
    e = jnp.exp(x - m)
    o_ref[...] = e / jnp.sum(e, axis=1, keepdims=True)


def _tc_softmax(p):
    n = p.shape[1]
    d = p.shape[2]
    rb = 1000
    return pl.pallas_call(
        _softmax_body,
        grid=(n // rb,),
        in_specs=[pl.BlockSpec((2, rb, d), lambda i: (0, i, 0))],
        out_specs=pl.BlockSpec((rb, d), lambda i: (i, 0)),
        out_shape=jax.ShapeDtypeStruct((n, d), jnp.float32),
    )(p)


def kernel(edge_index, rel_type, norm, W0, W1, W2):
    src = edge_index[0]
    dst = edge_index[1]
    nrm = norm[:, 0]

    pad = _E_PAD - _E
    srcm = jnp.pad(src, (0, pad)).reshape(_NW, _NCHUNK, _C)
    relm = jnp.pad(rel_type, (0, pad)).reshape(_NW, _NCHUNK, _C)
    dstm = jnp.pad(dst, (0, pad)).reshape(_NW, _NCHUNK, _C)
    metam = jnp.stack([srcm, relm, dstm], axis=2)
    normm = jnp.pad(nrm, (0, pad)).reshape(_NW, _EP)

    tab0 = W0.astype(jnp.bfloat16)[:, :, list(_PERM)].reshape(_R * _N, _H)
    p0 = _sc_pass(tab0, metam, normm, mode=0)[:, :_N]

    wc1 = W1.transpose(1, 0, 2)[:, :, list(_PERM)].reshape(_H, _R * _H)
    xw1 = _tc_relu_matmul(p0, wc1, out_dtype=jnp.bfloat16)
    p1 = _sc_pass(xw1.reshape(_N * _R, _H), metam, normm, mode=1)[:, :_N]

    xw2 = _tc_relu_matmul(p1, W2.transpose(1, 0, 2).reshape(_H, _R * _OUT))
    p2 = _sc_pass(xw2.reshape(_N * _R, _OUT), metam, normm, mode=1)[:, :_N]

    return _tc_softmax(p2)


# speedup vs baseline: 1.6192x; 1.6192x over previous
import functools

import jax
import jax.numpy as jnp
import numpy as np
from jax import lax
from jax.experimental import pallas as pl
from jax.experimental.pallas import tpu as pltpu
from jax.experimental.pallas import tpu_sc as plsc

_N = 10000
_H = 128
_OUT = 16
_R = 16
_E = 320000

_NC = 2
_NS = 16
_NW = _NC * _NS
_EP = 10240
_E_PAD = _NW * _EP
_C = 128
_NCHUNK = _EP // _C
_NPAD = 10240
_RPT = _NPAD // _NS
_ZR = 128
_NSLAB = 4
_PEEL = 4


def _sc_pass_body(mode, bf16_tab, d, table, metam, normm, out,
                  slabs, rows, scat_v, norm_v, acc_ref,
                  sem_m, sem_g0, sem_g1, sem_s):
    cid = lax.axis_index("c")
    sid = lax.axis_index("s")
    wid = sid * _NC + cid
    sem_g = (sem_g0, sem_g1)

    pltpu.sync_copy(normm.at[wid], norm_v)

    def _zero_row(r, _):
        for g in range(d // 16):
            scat_v[r, pl.ds(g * 16, 16)] = jnp.zeros((16,), jnp.float32)
        return 0
    lax.fori_loop(0, _ZR, _zero_row, 0)
    for j in range(_RPT // _ZR):
        pltpu.sync_copy(scat_v.at[pl.ds(0, _ZR)],
                        acc_ref.at[pl.ds(sid * _RPT + j * _ZR, _ZR)])
    plsc.subcore_barrier()

    def fire_meta(i, s):
        pltpu.async_copy(metam.at[wid, i], slabs[s], sem_m)

    def wait_meta(i, s):
        pltpu.make_async_copy(metam.at[wid, i], slabs[s], sem_m).wait()

    def idx_compute(s):
        for g in range(_C // 16):
            sl = pl.ds(g * 16, 16)
            sv = slabs[s][0, sl]
            rl = slabs[s][1, sl]
            if mode == 0:
                slabs[s][0, sl] = rl * _N + sv
            else:
                slabs[s][0, sl] = sv * _R + rl

    def fire_gather(s, b):
        pltpu.async_copy(table.at[slabs[s].at[0]], rows[b], sem_g[b])

    def wait_gather(s, b):
        pltpu.make_async_copy(table.at[slabs[s].at[0]], rows[b],
                              sem_g[b]).wait()

    def fire_scat(s):
        pltpu.async_copy(scat_v, acc_ref.at[slabs[s].at[2]], sem_s, add=True)

    def wait_scat(s):
        pltpu.make_async_copy(scat_v, acc_ref.at[slabs[s].at[2]],
                              sem_s).wait()

    def scale(i, b):
        def _scale(q, _):
            nv16 = norm_v[pl.ds(i * _C + q * 16, 16)]
            for l in range(16):
                nv = jnp.full((16,), nv16[l], jnp.float32)
                e = q * 16 + l
                if bf16_tab:
                    for g in range(d // 32):
                        vi = plsc.bitcast(rows[b][e, pl.ds(g * 32, 32)],
                                          jnp.int32)
                        lo = plsc.bitcast(vi << 16, jnp.float32)
                        hi = plsc.bitcast(vi & jnp.int32(-65536), jnp.float32)
                        scat_v[e, pl.ds(g * 32, 16)] = lo * nv
                        scat_v[e, pl.ds(g * 32 + 16, 16)] = hi * nv
                else:
                    for g in range(d // 16):
                        sl = pl.ds(g * 16, 16)
                        scat_v[e, sl] = rows[b][e, sl] * nv
            return 0
        lax.fori_loop(0, _C // 16, _scale, 0)

    def iteration(i, c, with_prev_scat, with_next1, with_next2):
        if with_next1:
            wait_meta(i + 1, (c + 1) % _NSLAB)
            idx_compute((c + 1) % _NSLAB)
            fire_gather((c + 1) % _NSLAB, (c + 1) % 2)
        if with_prev_scat:
            wait_scat((c - 1) % _NSLAB)
        if with_next2:
            fire_meta(i + 2, (c + 2) % _NSLAB)
        wait_gather(c % _NSLAB, c % 2)
        scale(i, c % 2)
        fire_scat(c % _NSLAB)

    fire_meta(0, 0)
    fire_meta(1, 1)
    wait_meta(0, 0)
    idx_compute(0)
    fire_gather(0, 0)
    for i in range(_PEEL):
        iteration(i, i, with_prev_scat=(i >= 1), with_next1=True,
                  with_next2=(i + 2 < _NCHUNK))

    def _main(j, _):
        for c in range(_PEEL):
            iteration(_PEEL + _PEEL * j + c, c, True, True, True)
        return 0
    lax.fori_loop(0, (_NCHUNK - 2 * _PEEL) // _PEEL, _main, 0)

    for i in range(_NCHUNK - _PEEL, _NCHUNK):
        iteration(i, i % _NSLAB, with_prev_scat=True,
                  with_next1=(i + 1 < _NCHUNK),
                  with_next2=(i + 2 < _NCHUNK))
    wait_scat((_NCHUNK - 1) % _NSLAB)

    plsc.subcore_barrier()
    pltpu.sync_copy(acc_ref.at[pl.ds(sid * _RPT, _RPT)],
                    out.at[cid, pl.ds(sid * _RPT, _RPT)])


def _sc_pass(table, metam, normm, mode):
    d = table.shape[1]
    bf16_tab = table.dtype == jnp.bfloat16
    body = functools.partial(_sc_pass_body, mode, bf16_tab, d)

    def wrapped(table, metam, normm, out, s0, s1, s2, s3, r0, r1,
                scat_v, norm_v, acc_ref, sem_m, sem_g0, sem_g1, sem_s):
        body(table, metam, normm, out, (s0, s1, s2, s3), (r0, r1),
             scat_v, norm_v, acc_ref, sem_m, sem_g0, sem_g1, sem_s)

    slab_t = pltpu.VMEM((3, _C), jnp.int32)
    return pl.kernel(
        wrapped,
        out_type=jax.ShapeDtypeStruct((_NC, _NPAD, d), jnp.float32),
        mesh=plsc.VectorSubcoreMesh(core_axis_name="c", subcore_axis_name="s"),
        scratch_types=[
            slab_t, slab_t, slab_t, slab_t,
            pltpu.VMEM((_C, d), table.dtype),
            pltpu.VMEM((_C, d), table.dtype),
            pltpu.VMEM((_C, d), jnp.float32),
            pltpu.VMEM((_EP,), jnp.float32),
            pltpu.VMEM_SHARED((_NPAD, d), jnp.float32),
            pltpu.SemaphoreType.DMA,
            pltpu.SemaphoreType.DMA,
            pltpu.SemaphoreType.DMA,
            pltpu.SemaphoreType.DMA,
        ],
        compiler_params=pltpu.CompilerParams(use_tc_tiling_on_sc=False,
                                             needs_layout_passes=False),
    )(table, metam, normm)


_PI = tuple(
    (np.arange(0, 128, 32)[:, None]
     + np.arange(32).reshape(16, 2).T.reshape(-1)[None, :]).reshape(-1))


def _mm_body(out_dtype, p_ref, w_ref, o_ref):
    h = jnp.maximum(p_ref[0] + p_ref[1], 0.0)
    o_ref[...] = jnp.dot(
        h, w_ref[...], preferred_element_type=jnp.float32).astype(out_dtype)


def _tc_relu_matmul(p, w, out_dtype=jnp.float32):
    n = p.shape[1]
    f = w.shape[1]
    rb = 400
    fb = min(f, 512)
    grid = (n // rb, f // fb)
    return pl.pallas_call(
        functools.partial(_mm_body, out_dtype),
        grid=grid,
        in_specs=[
            pl.BlockSpec((2, rb, _H), lambda i, j: (0, i, 0)),
            pl.BlockSpec((_H, fb), lambda i, j: (0, j)),
        ],
        out_specs=pl.BlockSpec((rb, fb), lambda i, j: (i, j)),
        out_shape=jax.ShapeDtypeStruct((n, f), out_dtype),
    )(p, w)


def _softmax_body(p_ref, o_ref):
    x = p_ref[0] + p_ref[1]
    m = jnp.max(x, axis=1, keepdims=True)
    e = jnp.exp(x - m)
    o_ref[...] = e / jnp.sum(e, axis=1, keepdims=True)


def _tc_softmax(p):
    n = p.shape[1]
    d = p.shape[2]
    rb = 1000
    return pl.pallas_call(
        _softmax_body,
        grid=(n // rb,),
        in_specs=[pl.BlockSpec((2, rb, d), lambda i: (0, i, 0))],
        out_specs=pl.BlockSpec((rb, d), lambda i: (i, 0)),
        out_shape=jax.ShapeDtypeStruct((n, d), jnp.float32),
    )(p)


def kernel(edge_index, rel_type, norm, W0, W1, W2):
    src = edge_index[0]
    dst = edge_index[1]
    nrm = norm[:, 0]

    pad = _E_PAD - _E
    srcm = jnp.pad(src, (0, pad)).reshape(_NW, _NCHUNK, _C)
    relm = jnp.pad(rel_type, (0, pad)).reshape(_NW, _NCHUNK, _C)
    dstm = jnp.pad(dst, (0, pad)).reshape(_NW, _NCHUNK, _C)
    metam = jnp.stack([srcm, relm, dstm], axis=2)
    normm = jnp.pad(nrm, (0, pad)).reshape(_NW, _EP)

    tab0 = W0.astype(jnp.bfloat16).reshape(_R * _N, _H)
    p0 = _sc_pass(tab0, metam, normm, mode=0)[:, :_N]

    wc1 = W1.transpose(1, 0, 2).reshape(_H, _R * _H)[np.array(_PI)]
    xw1 = _tc_relu_matmul(p0, wc1, out_dtype=jnp.bfloat16)
    p1 = _sc_pass(xw1.reshape(_N * _R, _H), metam, normm, mode=1)[:, :_N]

    wc2 = W2.transpose(1, 0, 2).reshape(_H, _R * _OUT)[np.array(_PI)]
    xw2 = _tc_relu_matmul(p1, wc2)
    p2 = _sc_pass(xw2.reshape(_N * _R, _OUT), metam, normm, mode=1)[:, :_N]

    return _tc_softmax(p2)


# speedup vs baseline: 1.6680x; 1.0302x over previous
import functools

import jax
import jax.numpy as jnp
import numpy as np
from jax import lax
from jax.experimental import pallas as pl
from jax.experimental.pallas import tpu as pltpu
from jax.experimental.pallas import tpu_sc as plsc

_N = 10000
_H = 128
_OUT = 16
_R = 16
_E = 320000

_NC = 2
_NS = 16
_NW = _NC * _NS
_EP = 10240
_E_PAD = _NW * _EP
_C = 128
_NCHUNK = _EP // _C
_NPAD = 10240
_RPT = _NPAD // _NS
_ZR = 128
_NSLAB = 4
_PEEL = 4


def _sc_pass_body(mode, bf16_tab, d, table, metam, normm, out,
                  slabs, rows, scat_v, norm_v, acc_ref,
                  sem_m, sem_g0, sem_g1, sem_s):
    cid = lax.axis_index("c")
    sid = lax.axis_index("s")
    wid = sid * _NC + cid
    sem_g = (sem_g0, sem_g1)

    pltpu.sync_copy(normm.at[wid], norm_v)

    def _zero_row(r, _):
        for g in range(d // 16):
            scat_v[r, pl.ds(g * 16, 16)] = jnp.zeros((16,), jnp.float32)
        return 0
    lax.fori_loop(0, _ZR, _zero_row, 0)
    for j in range(_RPT // _ZR):
        pltpu.sync_copy(scat_v.at[pl.ds(0, _ZR)],
                        acc_ref.at[pl.ds(sid * _RPT + j * _ZR, _ZR)])
    plsc.subcore_barrier()

    def fire_meta(i, s):
        pltpu.async_copy(metam.at[wid, i], slabs[s], sem_m)

    def wait_meta(i, s):
        pltpu.make_async_copy(metam.at[wid, i], slabs[s], sem_m).wait()

    def idx_compute(s):
        for g in range(_C // 16):
            sl = pl.ds(g * 16, 16)
            sv = slabs[s][0, sl]
            rl = slabs[s][1, sl]
            if mode == 0:
                slabs[s][0, sl] = rl * _N + sv
            else:
                slabs[s][0, sl] = sv * _R + rl

    def fire_gather(s, b):
        pltpu.async_copy(table.at[slabs[s].at[0]], rows[b], sem_g[b])

    def wait_gather(s, b):
        pltpu.make_async_copy(table.at[slabs[s].at[0]], rows[b],
                              sem_g[b]).wait()

    def fire_scat(s):
        pltpu.async_copy(scat_v, acc_ref.at[slabs[s].at[2]], sem_s, add=True)

    def wait_scat(s):
        pltpu.make_async_copy(scat_v, acc_ref.at[slabs[s].at[2]],
                              sem_s).wait()

    def scale(i, b):
        def _scale(q, _):
            nv16 = norm_v[pl.ds(i * _C + q * 16, 16)]
            for l in range(16):
                nv = jnp.full((16,), nv16[l], jnp.float32)
                e = q * 16 + l
                if bf16_tab:
                    for g in range(d // 32):
                        vi = plsc.bitcast(rows[b][e, pl.ds(g * 32, 32)],
                                          jnp.int32)
                        lo = plsc.bitcast(vi << 16, jnp.float32)
                        hi = plsc.bitcast(vi & jnp.int32(-65536), jnp.float32)
                        scat_v[e, pl.ds(g * 32, 16)] = lo * nv
                        scat_v[e, pl.ds(g * 32 + 16, 16)] = hi * nv
                else:
                    for g in range(d // 16):
                        sl = pl.ds(g * 16, 16)
                        scat_v[e, sl] = rows[b][e, sl] * nv
            return 0
        lax.fori_loop(0, _C // 16, _scale, 0)

    def iteration(i, c, with_prev_scat, with_next1, with_next2):
        if with_next1:
            wait_meta(i + 1, (c + 1) % _NSLAB)
            idx_compute((c + 1) % _NSLAB)
            fire_gather((c + 1) % _NSLAB, (c + 1) % 2)
        if with_prev_scat:
            wait_scat((c - 1) % _NSLAB)
        if with_next2:
            fire_meta(i + 2, (c + 2) % _NSLAB)
        wait_gather(c % _NSLAB, c % 2)
        scale(i, c % 2)
        fire_scat(c % _NSLAB)

    fire_meta(0, 0)
    fire_meta(1, 1)
    wait_meta(0, 0)
    idx_compute(0)
    fire_gather(0, 0)
    for i in range(_PEEL):
        iteration(i, i, with_prev_scat=(i >= 1), with_next1=True,
                  with_next2=(i + 2 < _NCHUNK))

    def _main(j, _):
        for c in range(_PEEL):
            iteration(_PEEL + _PEEL * j + c, c, True, True, True)
        return 0
    lax.fori_loop(0, (_NCHUNK - 2 * _PEEL) // _PEEL, _main, 0)

    for i in range(_NCHUNK - _PEEL, _NCHUNK):
        iteration(i, i % _NSLAB, with_prev_scat=True,
                  with_next1=(i + 1 < _NCHUNK),
                  with_next2=(i + 2 < _NCHUNK))
    wait_scat((_NCHUNK - 1) % _NSLAB)

    plsc.subcore_barrier()
    pltpu.sync_copy(acc_ref.at[pl.ds(sid * _RPT, _RPT)],
                    out.at[cid, pl.ds(sid * _RPT, _RPT)])


def _sc_pass(table, metam, normm, mode):
    d = table.shape[1]
    bf16_tab = table.dtype == jnp.bfloat16
    body = functools.partial(_sc_pass_body, mode, bf16_tab, d)

    def wrapped(table, metam, normm, out, s0, s1, s2, s3, r0, r1,
                scat_v, norm_v, acc_ref, sem_m, sem_g0, sem_g1, sem_s):
        body(table, metam, normm, out, (s0, s1, s2, s3), (r0, r1),
             scat_v, norm_v, acc_ref, sem_m, sem_g0, sem_g1, sem_s)

    slab_t = pltpu.VMEM((3, _C), jnp.int32)
    return pl.kernel(
        wrapped,
        out_type=jax.ShapeDtypeStruct((_NC, _NPAD, d), jnp.float32),
        mesh=plsc.VectorSubcoreMesh(core_axis_name="c", subcore_axis_name="s"),
        scratch_types=[
            slab_t, slab_t, slab_t, slab_t,
            pltpu.VMEM((_C, d), table.dtype),
            pltpu.VMEM((_C, d), table.dtype),
            pltpu.VMEM((_C, d), jnp.float32),
            pltpu.VMEM((_EP,), jnp.float32),
            pltpu.VMEM_SHARED((_NPAD, d), jnp.float32),
            pltpu.SemaphoreType.DMA,
            pltpu.SemaphoreType.DMA,
            pltpu.SemaphoreType.DMA,
            pltpu.SemaphoreType.DMA,
        ],
        compiler_params=pltpu.CompilerParams(use_tc_tiling_on_sc=False,
                                             needs_layout_passes=False),
    )(table, metam, normm)


_PI = tuple(
    (np.arange(0, 128, 32)[:, None]
     + np.arange(32).reshape(16, 2).T.reshape(-1)[None, :]).reshape(-1))


def _mm_body(out_dtype, p_ref, w_ref, o_ref):
    h = jnp.maximum(p_ref[0] + p_ref[1], 0.0)
    o_ref[...] = jnp.dot(
        h, w_ref[...], preferred_element_type=jnp.float32).astype(out_dtype)


def _tc_relu_matmul(p, w, out_dtype=jnp.float32):
    n = p.shape[1]
    f = w.shape[1]
    rb = 512 if n % 512 == 0 else 400
    fb = min(f, 512)
    grid = (n // rb, f // fb)
    return pl.pallas_call(
        functools.partial(_mm_body, out_dtype),
        grid=grid,
        in_specs=[
            pl.BlockSpec((2, rb, _H), lambda i, j: (0, i, 0)),
            pl.BlockSpec((_H, fb), lambda i, j: (0, j)),
        ],
        out_specs=pl.BlockSpec((rb, fb), lambda i, j: (i, j)),
        out_shape=jax.ShapeDtypeStruct((n, f), out_dtype),
    )(p, w)


def _softmax_body(p_ref, o_ref):
    x = p_ref[0] + p_ref[1]
    m = jnp.max(x, axis=1, keepdims=True)
    e = jnp.exp(x - m)
    o_ref[...] = e / jnp.sum(e, axis=1, keepdims=True)


def _tc_softmax(p):
    n = _N
    d = p.shape[2]
    rb = 1000
    return pl.pallas_call(
        _softmax_body,
        grid=(n // rb,),
        in_specs=[pl.BlockSpec((2, rb, d), lambda i: (0, i, 0))],
        out_specs=pl.BlockSpec((rb, d), lambda i: (i, 0)),
        out_shape=jax.ShapeDtypeStruct((n, d), jnp.float32),
    )(p)


def kernel(edge_index, rel_type, norm, W0, W1, W2):
    src = edge_index[0]
    dst = edge_index[1]
    nrm = norm[:, 0]

    pad = _E_PAD - _E
    srcm = jnp.pad(src, (0, pad)).reshape(_NW, _NCHUNK, _C)
    relm = jnp.pad(rel_type, (0, pad)).reshape(_NW, _NCHUNK, _C)
    dstm = jnp.pad(dst, (0, pad)).reshape(_NW, _NCHUNK, _C)
    metam = jnp.stack([srcm, relm, dstm], axis=2)
    normm = jnp.pad(nrm, (0, pad)).reshape(_NW, _EP)

    tab0 = W0.astype(jnp.bfloat16).reshape(_R * _N, _H)
    p0 = _sc_pass(tab0, metam, normm, mode=0)

    wc1 = W1.transpose(1, 0, 2).reshape(_H, _R * _H)[np.array(_PI)]
    xw1 = _tc_relu_matmul(p0, wc1, out_dtype=jnp.bfloat16)
    p1 = _sc_pass(xw1.reshape(_NPAD * _R, _H), metam, normm, mode=1)

    wc2 = W2.transpose(1, 0, 2).reshape(_H, _R * _OUT)[np.array(_PI)]
    xw2 = _tc_relu_matmul(p1, wc2)
    p2 = _sc_pass(xw2.reshape(_NPAD * _R, _OUT), metam, normm, mode=1)

    return _tc_softmax(p2)
